# baseline (speedup 1.0000x reference)
import functools

import jax
import jax.numpy as jnp
from jax import lax
from jax.experimental import pallas as pl
from jax.experimental.pallas import tpu as pltpu
from jax.experimental.pallas import tpu_sc as plsc

STACK_DIM = 1024
STACK_SIZE = 16384

_POINTER = 0
_READ_ROW = (_POINTER + 1) % STACK_SIZE


@functools.partial(
    pl.kernel,
    mesh=plsc.VectorSubcoreMesh(core_axis_name="c", subcore_axis_name="s"),
    out_type=jax.ShapeDtypeStruct((STACK_DIM,), jnp.float32),
    scratch_types=[pltpu.VMEM((STACK_DIM,), jnp.float32)],
)
def _pop_row(x_hbm, stack_hbm, out_hbm, row_vmem):
    nc = plsc.get_sparse_core_info().num_cores
    wid = lax.axis_index("s") * nc + lax.axis_index("c")

    @pl.when(wid == 0)
    def _():
        pltpu.sync_copy(stack_hbm.at[_READ_ROW], row_vmem)
        pltpu.sync_copy(row_vmem, out_hbm)


def kernel(x, stack):
    return _pop_row(x, stack)


# speedup vs baseline: 1.1835x; 1.1835x over previous
import functools

import jax
import jax.numpy as jnp
from jax import lax
from jax.experimental import pallas as pl
from jax.experimental.pallas import tpu as pltpu
from jax.experimental.pallas import tpu_sc as plsc

STACK_DIM = 1024
STACK_SIZE = 16384

_POINTER = 0
_READ_ROW = (_POINTER + 1) % STACK_SIZE


@functools.partial(
    pl.kernel,
    mesh=plsc.ScalarSubcoreMesh(axis_name="c", num_cores=1),
    out_type=jax.ShapeDtypeStruct((STACK_DIM,), jnp.float32),
)
def _pop_row(x_hbm, stack_hbm, out_hbm):
    pltpu.sync_copy(stack_hbm.at[_READ_ROW], out_hbm)


def kernel(x, stack):
    return _pop_row(x, stack)


# speedup vs baseline: 13.9138x; 11.7561x over previous
import jax
import jax.numpy as jnp
from jax.experimental import pallas as pl

STACK_DIM = 1024
STACK_SIZE = 16384

_POINTER = 0
_READ_ROW = (_POINTER + 1) % STACK_SIZE


def _copy_row(s_ref, o_ref):
    o_ref[...] = s_ref[_READ_ROW % 8, :]


def kernel(x, stack):
    return pl.pallas_call(
        _copy_row,
        out_shape=jax.ShapeDtypeStruct((STACK_DIM,), jnp.float32),
        grid=(1,),
        in_specs=[
            pl.BlockSpec((8, STACK_DIM), lambda i: (_READ_ROW // 8, 0)),
        ],
        out_specs=pl.BlockSpec((STACK_DIM,), lambda i: (0,)),
    )(stack)
